# baseline (speedup 1.0000x reference)
import functools

import jax
import jax.numpy as jnp
import numpy as np
from jax import lax
from jax.experimental import pallas as pl
from jax.experimental.pallas import tpu as pltpu
from jax.experimental.pallas import tpu_sc as plsc

LABEL_SMOOTHING = 0.1
VOCAB = 100000
CONFIDENCE = 1.0 - LABEL_SMOOTHING
BATCH = 1024
IGN = VOCAB - 100

NC, NS, L = 2, 16, 16
NW = NC * NS
B_PER_W = BATCH // NW
N_VREG = B_PER_W // L


def _sc_gather_body(flat_hbm, tgt_hbm, out_hbm, tgt_v, fit_v, fig_v,
                    gvt_v, gvi_v, st_v, sem):
    wid = lax.axis_index("s") * NC + lax.axis_index("c")
    base = wid * B_PER_W
    pltpu.sync_copy(tgt_hbm.at[pl.ds(base, B_PER_W)], tgt_v)
    for i in range(N_VREG):
        t16 = tgt_v[pl.ds(i * L, L)]
        rows = base + i * L + lax.iota(jnp.int32, L)
        fit_v[pl.ds(i * L, L)] = rows * VOCAB + t16
        fig_v[pl.ds(i * L, L)] = rows * VOCAB + IGN
    pltpu.async_copy(flat_hbm.at[fit_v], gvt_v, sem).wait()
    pltpu.async_copy(flat_hbm.at[fig_v], gvi_v, sem).wait()
    acc_g = jnp.zeros((L,), jnp.float32)
    acc_h = jnp.zeros((L,), jnp.float32)
    acc_c = jnp.zeros((L,), jnp.float32)
    for i in range(N_VREG):
        t16 = tgt_v[pl.ds(i * L, L)]
        is_ign = t16 == IGN
        acc_g = acc_g + gvt_v[pl.ds(i * L, L)]
        acc_h = acc_h + jnp.where(is_ign, 0.0, gvi_v[pl.ds(i * L, L)])
        acc_c = acc_c + jnp.where(is_ign, 1.0, 0.0)
    st_v[0, :] = acc_g
    st_v[1, :] = acc_h
    st_v[2, :] = acc_c
    pltpu.sync_copy(st_v, out_hbm.at[wid])


_sc_gather = functools.partial(
    pl.kernel,
    out_type=jax.ShapeDtypeStruct((NW, 3, L), jnp.float32),
    mesh=plsc.VectorSubcoreMesh(core_axis_name="c", subcore_axis_name="s"),
    scratch_types=[
        pltpu.VMEM((B_PER_W,), jnp.int32),
        pltpu.VMEM((B_PER_W,), jnp.int32),
        pltpu.VMEM((B_PER_W,), jnp.int32),
        pltpu.VMEM((B_PER_W,), jnp.float32),
        pltpu.VMEM((B_PER_W,), jnp.float32),
        pltpu.VMEM((3, L), jnp.float32),
        pltpu.SemaphoreType.DMA,
    ],
)(_sc_gather_body)


_SUM_R, _SUM_C = 8000, 12800
_SUM_BR = 160


def _tc_sum_body(x_ref, o_ref):
    @pl.when(pl.program_id(0) == 0)
    def _init():
        o_ref[...] = jnp.zeros_like(o_ref)

    o_ref[...] += jnp.sum(x_ref[...])[None, None]


def kernel(output, target):
    flat = output.reshape(-1)
    parts = _sc_gather(flat, target.astype(jnp.int32))
    g = jnp.sum(parts[:, 0, :])
    h = jnp.sum(parts[:, 1, :])
    cnt = jnp.sum(parts[:, 2, :])

    x2 = output.reshape(_SUM_R, _SUM_C)
    total = pl.pallas_call(
        _tc_sum_body,
        grid=(_SUM_R // _SUM_BR,),
        in_specs=[pl.BlockSpec((_SUM_BR, _SUM_C), lambda i: (i, 0))],
        out_specs=pl.BlockSpec((1, 1), lambda i: (0, 0)),
        out_shape=jax.ShapeDtypeStruct((1, 1), jnp.float32),
    )(x2)[0, 0]

    s = np.float32(LABEL_SMOOTHING / (VOCAB - 2))
    conf = np.float32(CONFIDENCE)
    plogp = (BATCH * conf * np.float32(np.log(CONFIDENCE))
             + s * np.float32(np.log(s)) * (BATCH * (VOCAB - 2) + cnt))
    return plogp - (s * total + (conf - s) * g - s * h)


# speedup vs baseline: 2.7737x; 2.7737x over previous
import functools

import jax
import jax.numpy as jnp
import numpy as np
from jax import lax
from jax.experimental import pallas as pl
from jax.experimental.pallas import tpu as pltpu
from jax.experimental.pallas import tpu_sc as plsc

LABEL_SMOOTHING = 0.1
VOCAB = 100000
CONFIDENCE = 1.0 - LABEL_SMOOTHING
BATCH = 1024
IGN = VOCAB - 100

NC, NS, L = 2, 16, 16
NW = NC * NS
B_PER_W = BATCH // NW
IGN_C0 = (IGN // L) * L
IGN_LANE = IGN - IGN_C0


TILE_R, TILE_C = 8, 128
TAIL0 = (VOCAB // TILE_C) * TILE_C
TAIL_W = VOCAB - TAIL0
LAST_FULL = TAIL0 - TILE_C
IGN_T0 = (IGN // TILE_C) * TILE_C
N_RB = B_PER_W // TILE_R


def _sc_gather_body(out_hbm, tgt_hbm, part_hbm, tgt_v, gbuf_v, tbuf_v,
                    ibuf_v, st_v, sem):
    wid = lax.axis_index("s") * NC + lax.axis_index("c")
    base = wid * B_PER_W
    pltpu.sync_copy(tgt_hbm.at[pl.ds(base, B_PER_W)], tgt_v)
    tvecs = [tgt_v[pl.ds(k * L, L)] for k in range(B_PER_W // L)]
    ts = [tvecs[i // L][i % L] for i in range(B_PER_W)]
    copies = []
    for k in range(N_RB):
        r0 = base + k * TILE_R
        copies.append(pltpu.async_copy(
            out_hbm.at[pl.ds(r0, TILE_R), pl.ds(TAIL0, TAIL_W)],
            tbuf_v.at[k], sem))
        copies.append(pltpu.async_copy(
            out_hbm.at[pl.ds(r0, TILE_R), pl.ds(IGN_T0, TILE_C)],
            ibuf_v.at[k], sem))
    for i in range(B_PER_W):
        t = ts[i]
        c0 = pl.multiple_of(
            jnp.minimum((t >> 7) << 7, LAST_FULL), TILE_C)
        r0 = base + (i // TILE_R) * TILE_R
        copies.append(pltpu.async_copy(
            out_hbm.at[pl.ds(r0, TILE_R), pl.ds(c0, TILE_C)],
            gbuf_v.at[i], sem))
    for c in copies:
        c.wait()
    lanes = lax.iota(jnp.int32, L)
    acc_g = jnp.zeros((L,), jnp.float32)
    acc_h = jnp.zeros((L,), jnp.float32)
    acc_c = jnp.zeros((L,), jnp.float32)
    for i in range(B_PER_W):
        t = ts[i]
        below_f = jnp.where(t < TAIL0, 1.0, 0.0)
        is_ign_f = jnp.where(t == IGN, 1.0, 0.0)
        off = t - jnp.minimum((t >> 7) << 7, LAST_FULL)
        sub = jnp.minimum((off >> 4) << 4, TILE_C - L)
        vm = gbuf_v[i, i % TILE_R, pl.ds(sub, L)]
        lane_m = jnp.full((L,), off - sub, jnp.int32)
        acc_g = acc_g + jnp.where(lanes == lane_m, vm, 0.0) * jnp.full(
            (L,), below_f, jnp.float32)
        offt = jnp.maximum(t - TAIL0, 0)
        subt = jnp.minimum((offt >> 4) << 4, TAIL_W - L)
        vt = tbuf_v[i // TILE_R, i % TILE_R, pl.ds(subt, L)]
        lane_t = jnp.full((L,), offt - subt, jnp.int32)
        acc_g = acc_g + jnp.where(lanes == lane_t, vt, 0.0) * jnp.full(
            (L,), 1.0 - below_f, jnp.float32)
        ign_sub = ((IGN - IGN_T0) // L) * L
        ign_lane = (IGN - IGN_T0) - ign_sub
        vi = ibuf_v[i // TILE_R, i % TILE_R, pl.ds(ign_sub, L)]
        acc_h = acc_h + jnp.where(lanes == ign_lane, vi, 0.0) * jnp.full(
            (L,), 1.0 - is_ign_f, jnp.float32)
        acc_c = acc_c + jnp.where(lanes == 0, jnp.full(
            (L,), is_ign_f, jnp.float32), 0.0)
    st_v[0, :] = acc_g
    st_v[1, :] = acc_h
    st_v[2, :] = acc_c
    pltpu.sync_copy(st_v, part_hbm.at[wid])


_sc_gather = functools.partial(
    pl.kernel,
    out_type=jax.ShapeDtypeStruct((NW, 3, L), jnp.float32),
    mesh=plsc.VectorSubcoreMesh(core_axis_name="c", subcore_axis_name="s"),
    scratch_types=[
        pltpu.VMEM((B_PER_W,), jnp.int32),
        pltpu.VMEM((B_PER_W, TILE_R, TILE_C), jnp.float32),
        pltpu.VMEM((N_RB, TILE_R, TAIL_W), jnp.float32),
        pltpu.VMEM((N_RB, TILE_R, TILE_C), jnp.float32),
        pltpu.VMEM((3, L), jnp.float32),
        pltpu.SemaphoreType.DMA,
    ],
)(_sc_gather_body)


_SUM_BR = 32


def _tc_sum_body(x_ref, o_ref):
    @pl.when(pl.program_id(0) == 0)
    def _init():
        o_ref[...] = jnp.zeros_like(o_ref)

    o_ref[...] += jnp.sum(x_ref[...])[None, None]


def kernel(output, target):
    parts = _sc_gather(output, target.astype(jnp.int32))
    g = jnp.sum(parts[:, 0, :])
    h = jnp.sum(parts[:, 1, :])
    cnt = jnp.sum(parts[:, 2, :])

    total = pl.pallas_call(
        _tc_sum_body,
        grid=(BATCH // _SUM_BR,),
        in_specs=[pl.BlockSpec((_SUM_BR, VOCAB), lambda i: (i, 0))],
        out_specs=pl.BlockSpec((1, 1), lambda i: (0, 0)),
        out_shape=jax.ShapeDtypeStruct((1, 1), jnp.float32),
    )(output)[0, 0]

    s = np.float32(LABEL_SMOOTHING / (VOCAB - 2))
    conf = np.float32(CONFIDENCE)
    plogp = (BATCH * conf * np.float32(np.log(CONFIDENCE))
             + s * np.float32(np.log(s)) * (BATCH * (VOCAB - 2) + cnt))
    return plogp - (s * total + (conf - s) * g - s * h)


# speedup vs baseline: 2.8029x; 1.0105x over previous
import functools

import jax
import jax.numpy as jnp
import numpy as np
from jax import lax
from jax.experimental import pallas as pl
from jax.experimental.pallas import tpu as pltpu
from jax.experimental.pallas import tpu_sc as plsc

LABEL_SMOOTHING = 0.1
VOCAB = 100000
CONFIDENCE = 1.0 - LABEL_SMOOTHING
BATCH = 1024
IGN = VOCAB - 100

NC, NS, L = 2, 16, 16
NW = NC * NS
B_PER_W = BATCH // NW
IGN_C0 = (IGN // L) * L
IGN_LANE = IGN - IGN_C0


TILE_R, TILE_C = 8, 128
TAIL0 = (VOCAB // TILE_C) * TILE_C
TAIL_W = VOCAB - TAIL0
LAST_FULL = TAIL0 - TILE_C
IGN_T0 = (IGN // TILE_C) * TILE_C
N_RB = B_PER_W // TILE_R


def _sc_gather_body(out_hbm, tgt_hbm, part_hbm, tgt_v, gbuf_v, tbuf_v,
                    ibuf_v, st_v, sem):
    wid = lax.axis_index("s") * NC + lax.axis_index("c")
    base = wid * B_PER_W
    pltpu.sync_copy(tgt_hbm.at[pl.ds(base, B_PER_W)], tgt_v)
    tvecs = [tgt_v[pl.ds(k * L, L)] for k in range(B_PER_W // L)]
    ts = [tvecs[i // L][i % L] for i in range(B_PER_W)]
    copies = []
    for k in range(N_RB):
        r0 = base + k * TILE_R
        copies.append(pltpu.async_copy(
            out_hbm.at[pl.ds(r0, TILE_R), pl.ds(TAIL0, TAIL_W)],
            tbuf_v.at[k], sem))
        copies.append(pltpu.async_copy(
            out_hbm.at[pl.ds(r0, TILE_R), pl.ds(IGN_T0, TILE_C)],
            ibuf_v.at[k], sem))
    for i in range(B_PER_W):
        t = ts[i]
        c0 = pl.multiple_of(
            jnp.minimum((t >> 7) << 7, LAST_FULL), TILE_C)
        r0 = base + (i // TILE_R) * TILE_R
        copies.append(pltpu.async_copy(
            out_hbm.at[pl.ds(r0, TILE_R), pl.ds(c0, TILE_C)],
            gbuf_v.at[i], sem))
    for c in copies:
        c.wait()
    lanes = lax.iota(jnp.int32, L)
    acc_g = jnp.zeros((L,), jnp.float32)
    acc_h = jnp.zeros((L,), jnp.float32)
    acc_c = jnp.zeros((L,), jnp.float32)
    for i in range(B_PER_W):
        t = ts[i]
        below_f = jnp.where(t < TAIL0, 1.0, 0.0)
        is_ign_f = jnp.where(t == IGN, 1.0, 0.0)
        off = t - jnp.minimum((t >> 7) << 7, LAST_FULL)
        sub = jnp.minimum((off >> 4) << 4, TILE_C - L)
        vm = gbuf_v[i, i % TILE_R, pl.ds(sub, L)]
        lane_m = jnp.full((L,), off - sub, jnp.int32)
        acc_g = acc_g + jnp.where(lanes == lane_m, vm, 0.0) * jnp.full(
            (L,), below_f, jnp.float32)
        offt = jnp.maximum(t - TAIL0, 0)
        subt = jnp.minimum((offt >> 4) << 4, TAIL_W - L)
        vt = tbuf_v[i // TILE_R, i % TILE_R, pl.ds(subt, L)]
        lane_t = jnp.full((L,), offt - subt, jnp.int32)
        acc_g = acc_g + jnp.where(lanes == lane_t, vt, 0.0) * jnp.full(
            (L,), 1.0 - below_f, jnp.float32)
        ign_sub = ((IGN - IGN_T0) // L) * L
        ign_lane = (IGN - IGN_T0) - ign_sub
        vi = ibuf_v[i // TILE_R, i % TILE_R, pl.ds(ign_sub, L)]
        acc_h = acc_h + jnp.where(lanes == ign_lane, vi, 0.0) * jnp.full(
            (L,), 1.0 - is_ign_f, jnp.float32)
        acc_c = acc_c + jnp.where(lanes == 0, jnp.full(
            (L,), is_ign_f, jnp.float32), 0.0)
    st_v[0, :] = acc_g
    st_v[1, :] = acc_h
    st_v[2, :] = acc_c
    pltpu.sync_copy(st_v, part_hbm.at[wid])


_sc_gather = functools.partial(
    pl.kernel,
    out_type=jax.ShapeDtypeStruct((NW, 3, L), jnp.float32),
    mesh=plsc.VectorSubcoreMesh(core_axis_name="c", subcore_axis_name="s"),
    scratch_types=[
        pltpu.VMEM((B_PER_W,), jnp.int32),
        pltpu.VMEM((B_PER_W, TILE_R, TILE_C), jnp.float32),
        pltpu.VMEM((N_RB, TILE_R, TAIL_W), jnp.float32),
        pltpu.VMEM((N_RB, TILE_R, TILE_C), jnp.float32),
        pltpu.VMEM((3, L), jnp.float32),
        pltpu.SemaphoreType.DMA,
    ],
)(_sc_gather_body)


_SUM_BR = 64


def _tc_sum_body(x_ref, o_ref):
    @pl.when(pl.program_id(0) == 0)
    def _init():
        o_ref[...] = jnp.zeros_like(o_ref)

    o_ref[...] += jnp.sum(x_ref[...])[None, None]


def kernel(output, target):
    parts = _sc_gather(output, target.astype(jnp.int32))
    g = jnp.sum(parts[:, 0, :])
    h = jnp.sum(parts[:, 1, :])
    cnt = jnp.sum(parts[:, 2, :])

    total = pl.pallas_call(
        _tc_sum_body,
        grid=(BATCH // _SUM_BR,),
        in_specs=[pl.BlockSpec((_SUM_BR, VOCAB), lambda i: (i, 0))],
        out_specs=pl.BlockSpec((1, 1), lambda i: (0, 0)),
        out_shape=jax.ShapeDtypeStruct((1, 1), jnp.float32),
    )(output)[0, 0]

    s = np.float32(LABEL_SMOOTHING / (VOCAB - 2))
    conf = np.float32(CONFIDENCE)
    plogp = (BATCH * conf * np.float32(np.log(CONFIDENCE))
             + s * np.float32(np.log(s)) * (BATCH * (VOCAB - 2) + cnt))
    return plogp - (s * total + (conf - s) * g - s * h)
